# baseline (speedup 1.0000x reference)
import functools

import jax
import jax.numpy as jnp
from jax import lax
from jax.experimental import pallas as pl
from jax.experimental.pallas import tpu as pltpu
from jax.experimental.pallas import tpu_sc as plsc

NC = 2
NS = 16
NW = NC * NS
LANES = 16

BATCH = 4
SEQ = 8192
DIM = 1024
POS_WORDS = SEQ * DIM
WORDS_W = POS_WORDS // NW
T_ROWS = 32
CH = T_ROWS * DIM
CHUNKS = WORDS_W // CH


def _sc_body(x_hbm, pos_hbm, out_hbm, xbuf, posbuf):
    wid = lax.axis_index("s") * NC + lax.axis_index("c")
    base = wid * WORDS_W
    for ci in range(CHUNKS):
        poff = base + ci * CH
        pltpu.sync_copy(pos_hbm.at[pl.ds(poff, CH)], posbuf)
        for b in range(BATCH):
            xoff = b * POS_WORDS + poff
            pltpu.sync_copy(x_hbm.at[pl.ds(xoff, CH)], xbuf)

            @plsc.parallel_loop(0, CH, step=LANES, unroll=8)
            def _(i):
                xbuf[pl.ds(i, LANES)] = (
                    xbuf[pl.ds(i, LANES)] + posbuf[pl.ds(i, LANES)]
                )

            pltpu.sync_copy(xbuf, out_hbm.at[pl.ds(xoff, CH)])


_sc_call = functools.partial(
    pl.kernel,
    out_type=jax.ShapeDtypeStruct((BATCH * POS_WORDS,), jnp.float32),
    mesh=plsc.VectorSubcoreMesh(core_axis_name="c", subcore_axis_name="s"),
    scratch_types=[
        pltpu.VMEM((CH,), jnp.float32),
        pltpu.VMEM((CH,), jnp.float32),
    ],
)(_sc_body)


def kernel(x, pos_table):
    batch, seq_len, dim = x.shape
    out = _sc_call(x.reshape(-1), pos_table[:seq_len].reshape(-1))
    return out.reshape(x.shape)


# speedup vs baseline: 1.1844x; 1.1844x over previous
import functools

import jax
import jax.numpy as jnp
from jax import lax
from jax.experimental import pallas as pl
from jax.experimental.pallas import tpu as pltpu
from jax.experimental.pallas import tpu_sc as plsc

NC = 2
NS = 16
NW = NC * NS
LANES = 16

BATCH = 4
SEQ = 8192
DIM = 1024
POS_WORDS = SEQ * DIM
WORDS_W = POS_WORDS // NW
T_ROWS = 16
CH = T_ROWS * DIM
CHUNKS = WORDS_W // CH
NPASS = CHUNKS * BATCH


def _sc_body(x_hbm, pos_hbm, out_hbm,
             xb0, xb1, pb0, pb1,
             xi0, xi1, xo0, xo1, ps0, ps1):
    xb = [xb0, xb1]
    pb = [pb0, pb1]
    xisem = [xi0, xi1]
    xosem = [xo0, xo1]
    psem = [ps0, ps1]

    wid = lax.axis_index("s") * NC + lax.axis_index("c")
    base = wid * WORDS_W

    def pos_off(ci):
        return base + ci * CH

    def x_off(p):
        ci, b = divmod(p, BATCH)
        return b * POS_WORDS + base + ci * CH

    pos_in = {}
    x_in = {}
    x_out = {}

    def issue_pos(ci):
        pos_in[ci] = pltpu.async_copy(
            pos_hbm.at[pl.ds(pos_off(ci), CH)], pb[ci % 2], psem[ci % 2])

    def issue_x_in(p):
        x_in[p] = pltpu.async_copy(
            x_hbm.at[pl.ds(x_off(p), CH)], xb[p % 2], xisem[p % 2])

    issue_pos(0)
    issue_x_in(0)
    for p in range(NPASS):
        ci, b = divmod(p, BATCH)
        k = p % 2
        if b == 0:
            pos_in[ci].wait()
            if ci + 1 < CHUNKS:
                issue_pos(ci + 1)
        if p + 1 < NPASS:
            if p - 1 >= 0:
                x_out[p - 1].wait()
            issue_x_in(p + 1)
        x_in[p].wait()
        pbuf = pb[ci % 2]
        xbuf = xb[k]

        @plsc.parallel_loop(0, CH, step=LANES, unroll=8)
        def _(i):
            xbuf[pl.ds(i, LANES)] = (
                xbuf[pl.ds(i, LANES)] + pbuf[pl.ds(i, LANES)]
            )

        x_out[p] = pltpu.async_copy(
            xbuf, out_hbm.at[pl.ds(x_off(p), CH)], xosem[k])
    x_out[NPASS - 2].wait()
    x_out[NPASS - 1].wait()


_sc_call = functools.partial(
    pl.kernel,
    out_type=jax.ShapeDtypeStruct((BATCH * POS_WORDS,), jnp.float32),
    mesh=plsc.VectorSubcoreMesh(core_axis_name="c", subcore_axis_name="s"),
    scratch_types=[
        pltpu.VMEM((CH,), jnp.float32),
        pltpu.VMEM((CH,), jnp.float32),
        pltpu.VMEM((CH,), jnp.float32),
        pltpu.VMEM((CH,), jnp.float32),
        pltpu.SemaphoreType.DMA,
        pltpu.SemaphoreType.DMA,
        pltpu.SemaphoreType.DMA,
        pltpu.SemaphoreType.DMA,
        pltpu.SemaphoreType.DMA,
        pltpu.SemaphoreType.DMA,
    ],
)(_sc_body)


def kernel(x, pos_table):
    batch, seq_len, dim = x.shape
    out = _sc_call(x.reshape(-1), pos_table[:seq_len].reshape(-1))
    return out.reshape(x.shape)


# speedup vs baseline: 3.1821x; 2.6867x over previous
import functools

import jax
import jax.numpy as jnp
from jax import lax
from jax.experimental import pallas as pl
from jax.experimental.pallas import tpu as pltpu
from jax.experimental.pallas import tpu_sc as plsc

NC = 2
NS = 16
NW = NC * NS
LANES = 16

BATCH = 4
SEQ = 8192
DIM = 1024
ROWS_W = SEQ // NW
T = 16
CHUNKS = ROWS_W // T
NPASS = CHUNKS * BATCH


def _sc_body(x_hbm, pos_hbm, out_hbm,
             xb0, xb1, pb0, pb1,
             xi0, xi1, xo0, xo1, ps0, ps1):
    xb = [xb0, xb1]
    pb = [pb0, pb1]
    xisem = [xi0, xi1]
    xosem = [xo0, xo1]
    psem = [ps0, ps1]

    wid = lax.axis_index("s") * NC + lax.axis_index("c")
    base = wid * ROWS_W

    def x_row(ci, b):
        return b * SEQ + base + ci * T

    def issue_x_in(ci, b, k):
        return pltpu.async_copy(
            x_hbm.at[pl.ds(x_row(ci, b), T)], xb[k], xisem[k])

    def issue_pos(ci, k):
        return pltpu.async_copy(
            pos_hbm.at[pl.ds(base + ci * T, T)], pb[k], psem[k])

    def wait_x_in(k):
        pltpu.make_async_copy(x_hbm.at[pl.ds(0, T)], xb[k], xisem[k]).wait()

    def wait_x_out(k):
        pltpu.make_async_copy(xb[k], out_hbm.at[pl.ds(0, T)], xosem[k]).wait()

    def wait_pos(k):
        pltpu.make_async_copy(pos_hbm.at[pl.ds(0, T)], pb[k], psem[k]).wait()

    issue_pos(0, 0)
    issue_x_in(0, 0, 0)

    last_ci = CHUNKS - 1

    @pl.loop(0, CHUNKS, step=2)
    def _(ci0):
        for dci in (0, 1):
            ci = ci0 + dci
            wait_pos(dci)
            issue_pos(jnp.minimum(ci + 1, last_ci), 1 - dci)
            for b in range(BATCH):
                k = b % 2
                if dci == 0 and b == 0:
                    @pl.when(ci0 > 0)
                    def _():
                        wait_x_out(1 - k)
                else:
                    wait_x_out(1 - k)
                if b + 1 < BATCH:
                    issue_x_in(ci, b + 1, 1 - k)
                else:
                    issue_x_in(jnp.minimum(ci + 1, last_ci), 0, 1 - k)
                wait_x_in(k)
                xbuf = xb[k]
                pbuf = pb[dci]

                @plsc.parallel_loop(0, T, 1)
                def _(r):
                    for j in range(DIM // LANES):
                        c = j * LANES
                        xbuf[r, pl.ds(c, LANES)] = (
                            xbuf[r, pl.ds(c, LANES)]
                            + pbuf[r, pl.ds(c, LANES)]
                        )

                pltpu.async_copy(
                    xbuf, out_hbm.at[pl.ds(x_row(ci, b), T)], xosem[k])

    wait_x_out((NPASS - 1) % 2)
    wait_x_in(NPASS % 2)
    wait_pos(CHUNKS % 2)


_sc_call = functools.partial(
    pl.kernel,
    out_type=jax.ShapeDtypeStruct((BATCH * SEQ, DIM), jnp.float32),
    mesh=plsc.VectorSubcoreMesh(core_axis_name="c", subcore_axis_name="s"),
    scratch_types=[
        pltpu.VMEM((T, DIM), jnp.float32),
        pltpu.VMEM((T, DIM), jnp.float32),
        pltpu.VMEM((T, DIM), jnp.float32),
        pltpu.VMEM((T, DIM), jnp.float32),
        pltpu.SemaphoreType.DMA,
        pltpu.SemaphoreType.DMA,
        pltpu.SemaphoreType.DMA,
        pltpu.SemaphoreType.DMA,
        pltpu.SemaphoreType.DMA,
        pltpu.SemaphoreType.DMA,
    ],
)(_sc_body)


def kernel(x, pos_table):
    batch, seq_len, dim = x.shape
    out = _sc_call(x.reshape(batch * seq_len, dim), pos_table[:seq_len])
    return out.reshape(x.shape)


# speedup vs baseline: 3.2413x; 1.0186x over previous
import functools

import jax
import jax.numpy as jnp
from jax import lax
from jax.experimental import pallas as pl
from jax.experimental.pallas import tpu as pltpu
from jax.experimental.pallas import tpu_sc as plsc

NC = 2
NS = 16
NW = NC * NS
LANES = 16

BATCH = 4
SEQ = 8192
DIM = 1024
ROWS_W = SEQ // NW
T = 8
CHUNKS = ROWS_W // T


def _sc_body(x_hbm, pos_hbm, out_hbm, *refs):
    xb = [[refs[2 * b + q] for q in (0, 1)] for b in range(BATCH)]
    pb = [refs[8], refs[9]]
    xisem = [[refs[10 + 2 * b + q] for q in (0, 1)] for b in range(BATCH)]
    xosem = [[refs[18 + 2 * b + q] for q in (0, 1)] for b in range(BATCH)]
    psem = [refs[26], refs[27]]

    wid = lax.axis_index("s") * NC + lax.axis_index("c")
    base = wid * ROWS_W
    last_ci = CHUNKS - 1

    def x_row(ci, b):
        return b * SEQ + base + ci * T

    def issue_x_in(ci, b, q):
        pltpu.async_copy(
            x_hbm.at[pl.ds(x_row(ci, b), T)], xb[b][q], xisem[b][q])

    def issue_x_out(ci, b, q):
        pltpu.async_copy(
            xb[b][q], out_hbm.at[pl.ds(x_row(ci, b), T)], xosem[b][q])

    def issue_pos(ci, q):
        pltpu.async_copy(
            pos_hbm.at[pl.ds(base + ci * T, T)], pb[q], psem[q])

    def wait_x_in(b, q):
        pltpu.make_async_copy(
            x_hbm.at[pl.ds(0, T)], xb[b][q], xisem[b][q]).wait()

    def wait_x_out(b, q):
        pltpu.make_async_copy(
            xb[b][q], out_hbm.at[pl.ds(0, T)], xosem[b][q]).wait()

    def wait_pos(q):
        pltpu.make_async_copy(
            pos_hbm.at[pl.ds(0, T)], pb[q], psem[q]).wait()

    issue_pos(0, 0)
    for b in range(BATCH):
        issue_x_in(0, b, 0)

    @pl.loop(0, CHUNKS, step=2)
    def _(ci0):
        for q in (0, 1):
            ci = ci0 + q
            ci_next = jnp.minimum(ci + 1, last_ci)
            wait_pos(q)
            issue_pos(ci_next, 1 - q)
            for b in range(BATCH):
                if q == 0 and b == 0:
                    @pl.when(ci0 > 0)
                    def _():
                        for bb in range(BATCH):
                            wait_x_out(bb, 1)
                elif q == 1:
                    wait_x_out(b, 0)
            for b in range(BATCH):
                issue_x_in(ci_next, b, 1 - q)
            for b in range(BATCH):
                wait_x_in(b, q)
            pbuf = pb[q]
            xcur = [xb[b][q] for b in range(BATCH)]

            @plsc.parallel_loop(0, T, 1)
            def _(r):
                for j in range(DIM // LANES):
                    c = j * LANES
                    pv = pbuf[r, pl.ds(c, LANES)]
                    for b in range(BATCH):
                        xcur[b][r, pl.ds(c, LANES)] = (
                            xcur[b][r, pl.ds(c, LANES)] + pv
                        )

            for b in range(BATCH):
                issue_x_out(ci, b, q)

    for b in range(BATCH):
        wait_x_out(b, (CHUNKS - 1) % 2)
        wait_x_in(b, CHUNKS % 2)
    wait_pos(CHUNKS % 2)


_sc_call = functools.partial(
    pl.kernel,
    out_type=jax.ShapeDtypeStruct((BATCH * SEQ, DIM), jnp.float32),
    mesh=plsc.VectorSubcoreMesh(core_axis_name="c", subcore_axis_name="s"),
    scratch_types=(
        [pltpu.VMEM((T, DIM), jnp.float32) for _ in range(10)]
        + [pltpu.SemaphoreType.DMA for _ in range(18)]
    ),
)(_sc_body)


def kernel(x, pos_table):
    batch, seq_len, dim = x.shape
    out = _sc_call(x.reshape(batch * seq_len, dim), pos_table[:seq_len])
    return out.reshape(x.shape)
